# baseline (speedup 1.0000x reference)
import functools

import jax
import jax.numpy as jnp
from jax import lax
from jax.experimental import pallas as pl
from jax.experimental.pallas import tpu as pltpu
from jax.experimental.pallas import tpu_sc as plsc

_N = 100000
_E = 3200000
_CG = 4
_BLK = 256
_K = _BLK // 128
_EPAD = 3211264
_PADN = 352
_NACC = _N + _PADN
_ZCH = 784
_RB = 6256
_NOUT = 16 * _RB
_NB = 2000


def _pad64(v):
    return jnp.pad(v, ((0, 0), (0, 64 - v.shape[1])))


def _m1_body(x_ref, w_ref, d_ref, h_ref, g_ref, dis_ref):
    deg = d_ref[:, 0:1] + 1.0
    dis = lax.rsqrt(deg)
    h = lax.dot_general(x_ref[...], w_ref[...], (((1,), (0,)), ((), ())),
                        precision=lax.Precision.HIGHEST,
                        preferred_element_type=jnp.float32)
    h_ref[...] = h
    g_ref[...] = _pad64(h * dis)
    dis_ref[...] = dis


def _layer1(x, w, d):
    cin, cout = w.shape
    grid = (_N // _NB,)
    return pl.pallas_call(
        _m1_body,
        grid=grid,
        in_specs=[
            pl.BlockSpec((_NB, cin), lambda i: (i, 0)),
            pl.BlockSpec((cin, cout), lambda i: (0, 0)),
            pl.BlockSpec((_NB, 16), lambda i: (i, 0)),
        ],
        out_specs=[
            pl.BlockSpec((_NB, cout), lambda i: (i, 0)),
            pl.BlockSpec((_NB, 64), lambda i: (i, 0)),
            pl.BlockSpec((_NB, 1), lambda i: (i, 0)),
        ],
        out_shape=[
            jax.ShapeDtypeStruct((_N, cout), jnp.float32),
            jax.ShapeDtypeStruct((_N, 64), jnp.float32),
            jax.ShapeDtypeStruct((_N, 1), jnp.float32),
        ],
    )(x, w, d)


def _fmid_body(agg_ref, h_ref, dis_ref, b_ref, w_ref, ho_ref, *g_refs):
    dis = dis_ref[...]
    xn = dis * agg_ref[...] + (dis * dis) * h_ref[...] + b_ref[...]
    xn = jnp.maximum(xn, 0.0)
    h2 = lax.dot_general(xn, w_ref[...], (((1,), (0,)), ((), ())),
                         precision=lax.Precision.HIGHEST,
                         preferred_element_type=jnp.float32)
    ho_ref[...] = h2
    g = h2 * dis
    if len(g_refs) == 1:
        g_refs[0][...] = _pad64(g)
    else:
        g_refs[0][...] = g[:, :64]
        g_refs[1][...] = g[:, 64:]


def _layer_mid(agg, h, dis, b, w):
    cin, cout = w.shape
    ngo = 2 if cout == 128 else 1
    grid = (_N // _NB,)
    return pl.pallas_call(
        _fmid_body,
        grid=grid,
        in_specs=[
            pl.BlockSpec((_NB, cin), lambda i: (i, 0)),
            pl.BlockSpec((_NB, cin), lambda i: (i, 0)),
            pl.BlockSpec((_NB, 1), lambda i: (i, 0)),
            pl.BlockSpec((1, cin), lambda i: (0, 0)),
            pl.BlockSpec((cin, cout), lambda i: (0, 0)),
        ],
        out_specs=[pl.BlockSpec((_NB, cout), lambda i: (i, 0))]
        + [pl.BlockSpec((_NB, 64), lambda i: (i, 0))] * ngo,
        out_shape=[jax.ShapeDtypeStruct((_N, cout), jnp.float32)]
        + [jax.ShapeDtypeStruct((_N, 64), jnp.float32)] * ngo,
    )(agg, h, dis, b, w)


def _ffin_body(agg_ref, h_ref, dis_ref, b_ref, o_ref):
    dis = dis_ref[...]
    o_ref[...] = dis * agg_ref[...] + (dis * dis) * h_ref[...] + b_ref[...]


def _layer_fin(agg, h, dis, b):
    c = agg.shape[1]
    grid = (_N // _NB,)
    return pl.pallas_call(
        _ffin_body,
        grid=grid,
        in_specs=[
            pl.BlockSpec((_NB, c), lambda i: (i, 0)),
            pl.BlockSpec((_NB, c), lambda i: (i, 0)),
            pl.BlockSpec((_NB, 1), lambda i: (i, 0)),
            pl.BlockSpec((1, c), lambda i: (0, 0)),
        ],
        out_specs=pl.BlockSpec((_NB, c), lambda i: (i, 0)),
        out_shape=jax.ShapeDtypeStruct((_N, c), jnp.float32),
    )(agg, h, dis, b)


_MESH = plsc.VectorSubcoreMesh(core_axis_name="c", subcore_axis_name="s")
_CGH = _CG // 2
_EPT = _EPAD // 16
_ST = _EPT // _BLK


def _make_agg():
    @functools.partial(
        pl.kernel, mesh=_MESH,
        out_type=(jax.ShapeDtypeStruct((_CGH, _NOUT, 16), jnp.float32),
                  jax.ShapeDtypeStruct((_CGH, _NOUT, 16), jnp.float32)),
        scratch_types=[
            pltpu.VMEM_SHARED((_NACC, 16), jnp.float32),
            pltpu.VMEM((4, 128), jnp.int32),
            pltpu.VMEM((4, 128), jnp.int32),
            pltpu.VMEM((_BLK,), jnp.int32),
            pltpu.VMEM((_BLK,), jnp.int32),
            pltpu.VMEM((_BLK, 16), jnp.float32),
            pltpu.VMEM((_BLK, 16), jnp.float32),
            pltpu.SemaphoreType.DMA,
            pltpu.SemaphoreType.DMA,
            pltpu.SemaphoreType.DMA,
            pltpu.SemaphoreType.DMA,
            pltpu.SemaphoreType.DMA,
            pltpu.SemaphoreType.DMA,
        ],
        compiler_params=pltpu.CompilerParams(use_tc_tiling_on_sc=False),
    )
    def k(g_hbm, edge_hbm, z_hbm, outa, outb,
          acc, edgev0, edgev1, gidx0, gidx1, rows0, rows1,
          sl0, sl1, sg0, sg1, ss0, ss1):
        c = lax.axis_index("c")
        s = lax.axis_index("s")
        bufs = [(edgev0, gidx0, rows0, sl0, sg0, ss0),
                (edgev1, gidx1, rows1, sl1, sg1, ss1)]

        def front(t, b, gg):
            edgev, gidx, rows, sl, sg, ss = bufs[b]
            erow = s * (_EPT // _BLK) + t
            pltpu.async_copy(edge_hbm.at[erow], edgev, sl).wait()
            for i in range(_BLK // 16):
                slc = pl.ds(i * 16, 16)
                esl = pl.ds((i % 8) * 16, 16)
                gidx[slc] = edgev[i // 8, esl] * _CG + gg
            pltpu.async_copy(g_hbm.at[gidx], rows, sg)

        def back(b):
            edgev, gidx, rows, sl, sg, ss = bufs[b]
            pltpu.make_async_copy(g_hbm.at[gidx], rows, sg).wait()
            for jj in range(_K):
                pltpu.async_copy(rows.at[pl.ds(jj * 128, 128)],
                                 acc.at[edgev.at[2 + jj]], ss, add=True)

        def drain(b):
            edgev, gidx, rows, sl, sg, ss = bufs[b]
            for jj in range(_K):
                pltpu.make_async_copy(rows.at[pl.ds(jj * 128, 128)],
                                      acc.at[edgev.at[2 + jj]], ss).wait()

        for j in range(_CGH):
            gg = c * _CGH + j

            zh = []
            for kk in range(8):
                zoff = pl.multiple_of((s * 8 + kk) * _ZCH, 8)
                zh.append(pltpu.async_copy(
                    z_hbm, acc.at[pl.ds(zoff, _ZCH)], sg0))
            for hd in zh:
                hd.wait()
            plsc.subcore_barrier()

            for b in range(2):
                front(b, b, gg)
            for b in range(2):
                back(b)

            def pbody(p, carry):
                for b in range(2):
                    drain(b)
                    front(2 * p + b, b, gg)
                for b in range(2):
                    back(b)
                return carry
            lax.fori_loop(1, _ST // 2, pbody, 0)
            for b in range(2):
                drain(b)
            plsc.subcore_barrier()

            woff = pl.multiple_of(s * _RB, 8)

            @pl.when(c == 0)
            def _():
                pltpu.sync_copy(acc.at[pl.ds(woff, _RB)],
                                outa.at[j, pl.ds(woff, _RB)])

            @pl.when(c == 1)
            def _():
                pltpu.sync_copy(acc.at[pl.ds(woff, _RB)],
                                outb.at[j, pl.ds(woff, _RB)])
            plsc.subcore_barrier()

    return k


def _aggregate(g64, edge4, zeros_h):
    outa, outb = _make_agg()(g64.reshape(_N * _CG, 16), edge4, zeros_h)
    agg_gm = jnp.concatenate([outa, outb], axis=0)[:, :_N]
    return agg_gm.transpose(1, 0, 2).reshape(_N, 64)


def kernel(x, edge_index, W1, b1, W2, b2, W3, b3, W4, b4):
    w4p = jnp.pad(W4, ((0, 0), (0, 11)))
    b4p = jnp.pad(b4, (0, 11))

    src = edge_index[0]
    dst = edge_index[1]
    padlen = _EPAD - _E
    ar = jnp.arange(padlen, dtype=jnp.int32)
    src1d = jnp.concatenate([src, ar % _N]).reshape(_EPAD // _BLK, 2, 128)
    dst1d = jnp.concatenate([dst, _N + (ar % _PADN)]).reshape(_EPAD // _BLK, 2, 128)
    edge4 = jnp.concatenate([src1d, dst1d], axis=1)

    zeros_h = jnp.zeros((_ZCH, 16), jnp.float32)
    ones_g = jnp.ones((_N, 64), jnp.float32)

    deg = _aggregate(ones_g, edge4, zeros_h)[:, 0:16]

    h1, g1, dis = _layer1(x, W1, deg)
    agg1 = _aggregate(g1, edge4, zeros_h)[:, :32]
    h2, g2 = _layer_mid(agg1, h1, dis, b1.reshape(1, -1), W2)
    agg2 = _aggregate(g2, edge4, zeros_h)
    h3, g3a, g3b = _layer_mid(agg2, h2, dis, b2.reshape(1, -1), W3)
    agg3 = jnp.concatenate(
        [_aggregate(g3a, edge4, zeros_h),
         _aggregate(g3b, edge4, zeros_h)], axis=1)
    h4, g4 = _layer_mid(agg3, h3, dis, b3.reshape(1, -1), w4p)
    agg4 = _aggregate(g4, edge4, zeros_h)[:, :32]
    out = _layer_fin(agg4, h4, dis, b4p.reshape(1, -1))
    return out[:, :21]


# speedup vs baseline: 1.4472x; 1.4472x over previous
import functools

import jax
import jax.numpy as jnp
from jax import lax
from jax.experimental import pallas as pl
from jax.experimental.pallas import tpu as pltpu
from jax.experimental.pallas import tpu_sc as plsc

_N = 100000
_E = 3200000
_CG = 4
_BLK = 256
_K = _BLK // 128
_EPAD = 3211264
_PADN = 352
_NACC = _N + _PADN
_ZCH = 784
_RB = 6256
_NOUT = 16 * _RB
_NB = 2000


def _pad64(v):
    return jnp.pad(v, ((0, 0), (0, 64 - v.shape[1])))


def _m1_body(x_ref, w_ref, d_ref, h_ref, g_ref, dis_ref):
    deg = d_ref[:, 0:1] + 1.0
    dis = lax.rsqrt(deg)
    h = lax.dot_general(x_ref[...], w_ref[...], (((1,), (0,)), ((), ())),
                        precision=lax.Precision.HIGHEST,
                        preferred_element_type=jnp.float32)
    h_ref[...] = h
    g_ref[...] = _pad64(h * dis)
    dis_ref[...] = dis


def _layer1(x, w, d):
    cin, cout = w.shape
    grid = (_N // _NB,)
    return pl.pallas_call(
        _m1_body,
        grid=grid,
        in_specs=[
            pl.BlockSpec((_NB, cin), lambda i: (i, 0)),
            pl.BlockSpec((cin, cout), lambda i: (0, 0)),
            pl.BlockSpec((_NB, 16), lambda i: (i, 0)),
        ],
        out_specs=[
            pl.BlockSpec((_NB, cout), lambda i: (i, 0)),
            pl.BlockSpec((_NB, 64), lambda i: (i, 0)),
            pl.BlockSpec((_NB, 1), lambda i: (i, 0)),
        ],
        out_shape=[
            jax.ShapeDtypeStruct((_N, cout), jnp.float32),
            jax.ShapeDtypeStruct((_N, 64), jnp.float32),
            jax.ShapeDtypeStruct((_N, 1), jnp.float32),
        ],
    )(x, w, d)


def _fmid_body(agg_ref, h_ref, dis_ref, b_ref, w_ref, ho_ref, *g_refs):
    dis = dis_ref[...]
    xn = dis * agg_ref[...] + (dis * dis) * h_ref[...] + b_ref[...]
    xn = jnp.maximum(xn, 0.0)
    h2 = lax.dot_general(xn, w_ref[...], (((1,), (0,)), ((), ())),
                         precision=lax.Precision.HIGHEST,
                         preferred_element_type=jnp.float32)
    ho_ref[...] = h2
    g = h2 * dis
    if len(g_refs) == 1:
        g_refs[0][...] = _pad64(g)
    else:
        g_refs[0][...] = g[:, :64]
        g_refs[1][...] = g[:, 64:]


def _layer_mid(agg, h, dis, b, w):
    cin, cout = w.shape
    ngo = 2 if cout == 128 else 1
    grid = (_N // _NB,)
    return pl.pallas_call(
        _fmid_body,
        grid=grid,
        in_specs=[
            pl.BlockSpec((_NB, cin), lambda i: (i, 0)),
            pl.BlockSpec((_NB, cin), lambda i: (i, 0)),
            pl.BlockSpec((_NB, 1), lambda i: (i, 0)),
            pl.BlockSpec((1, cin), lambda i: (0, 0)),
            pl.BlockSpec((cin, cout), lambda i: (0, 0)),
        ],
        out_specs=[pl.BlockSpec((_NB, cout), lambda i: (i, 0))]
        + [pl.BlockSpec((_NB, 64), lambda i: (i, 0))] * ngo,
        out_shape=[jax.ShapeDtypeStruct((_N, cout), jnp.float32)]
        + [jax.ShapeDtypeStruct((_N, 64), jnp.float32)] * ngo,
    )(agg, h, dis, b, w)


def _ffin_body(agg_ref, h_ref, dis_ref, b_ref, o_ref):
    dis = dis_ref[...]
    o_ref[...] = dis * agg_ref[...] + (dis * dis) * h_ref[...] + b_ref[...]


def _layer_fin(agg, h, dis, b):
    c = agg.shape[1]
    grid = (_N // _NB,)
    return pl.pallas_call(
        _ffin_body,
        grid=grid,
        in_specs=[
            pl.BlockSpec((_NB, c), lambda i: (i, 0)),
            pl.BlockSpec((_NB, c), lambda i: (i, 0)),
            pl.BlockSpec((_NB, 1), lambda i: (i, 0)),
            pl.BlockSpec((1, c), lambda i: (0, 0)),
        ],
        out_specs=pl.BlockSpec((_NB, c), lambda i: (i, 0)),
        out_shape=jax.ShapeDtypeStruct((_N, c), jnp.float32),
    )(agg, h, dis, b)


_MESH = plsc.VectorSubcoreMesh(core_axis_name="c", subcore_axis_name="s")
_CGH = _CG // 2
_EPT = _EPAD // 16
_ST = _EPT // _BLK


def _make_agg():
    @functools.partial(
        pl.kernel, mesh=_MESH,
        out_type=(jax.ShapeDtypeStruct((_CGH, _NOUT, 16), jnp.float32),
                  jax.ShapeDtypeStruct((_CGH, _NOUT, 16), jnp.float32)),
        scratch_types=[
            pltpu.VMEM_SHARED((_NACC, 16), jnp.float32),
            pltpu.VMEM((4, 128), jnp.int32),
            pltpu.VMEM((4, 128), jnp.int32),
            pltpu.VMEM((_K, 128), jnp.int32),
            pltpu.VMEM((_K, 128), jnp.int32),
            pltpu.VMEM((_BLK,), jnp.int32),
            pltpu.VMEM((_BLK,), jnp.int32),
            pltpu.VMEM((_BLK, 16), jnp.float32),
            pltpu.VMEM((_BLK, 16), jnp.float32),
            pltpu.SemaphoreType.DMA,
            pltpu.SemaphoreType.DMA,
            pltpu.SemaphoreType.DMA,
            pltpu.SemaphoreType.DMA,
            pltpu.SemaphoreType.DMA,
            pltpu.SemaphoreType.DMA,
        ],
        compiler_params=pltpu.CompilerParams(use_tc_tiling_on_sc=False),
    )
    def k(g_hbm, edge_hbm, z_hbm, outa, outb,
          acc, edgev0, edgev1, dstv0, dstv1, gidx0, gidx1, rows0, rows1,
          sl0, sl1, sg0, sg1, ss0, ss1):
        c = lax.axis_index("c")
        s = lax.axis_index("s")
        bufs = [(edgev0, dstv0, gidx0, rows0, sl0, sg0, ss0),
                (edgev1, dstv1, gidx1, rows1, sl1, sg1, ss1)]

        def fire_load(t, b):
            edgev = bufs[b][0]
            sl = bufs[b][4]
            erow = s * _ST + jnp.minimum(t, _ST - 1)
            pltpu.async_copy(edge_hbm.at[erow], edgev, sl)

        def front(t, b, gg):
            edgev, dstv, gidx, rows, sl, sg, ss = bufs[b]
            erow = s * _ST + jnp.minimum(t, _ST - 1)
            pltpu.make_async_copy(edge_hbm.at[erow], edgev, sl).wait()
            for i in range(_BLK // 16):
                slc = pl.ds(i * 16, 16)
                esl = pl.ds((i % 8) * 16, 16)
                gidx[slc] = edgev[i // 8, esl] * _CG + gg
            pltpu.async_copy(g_hbm.at[gidx], rows, sg)

        def back(t, b):
            edgev, dstv, gidx, rows, sl, sg, ss = bufs[b]
            for i in range(_K * 8):
                esl = pl.ds((i % 8) * 16, 16)
                dstv[i // 8, esl] = edgev[2 + i // 8, esl]
            fire_load(t + 2, b)
            pltpu.make_async_copy(g_hbm.at[gidx], rows, sg).wait()
            for jj in range(_K):
                pltpu.async_copy(rows.at[pl.ds(jj * 128, 128)],
                                 acc.at[dstv.at[jj]], ss, add=True)

        def drain(b):
            edgev, dstv, gidx, rows, sl, sg, ss = bufs[b]
            for jj in range(_K):
                pltpu.make_async_copy(rows.at[pl.ds(jj * 128, 128)],
                                      acc.at[dstv.at[jj]], ss).wait()

        for j in range(_CGH):
            gg = c * _CGH + j

            zh = []
            for kk in range(8):
                zoff = pl.multiple_of((s * 8 + kk) * _ZCH, 8)
                zh.append(pltpu.async_copy(
                    z_hbm, acc.at[pl.ds(zoff, _ZCH)], sg0))
            for hd in zh:
                hd.wait()
            plsc.subcore_barrier()

            for b in range(2):
                fire_load(b, b)
            for b in range(2):
                front(b, b, gg)
            for b in range(2):
                back(b, b)

            def pbody(p, carry):
                for b in range(2):
                    drain(b)
                    front(2 * p + b, b, gg)
                for b in range(2):
                    back(2 * p + b, b)
                return carry
            lax.fori_loop(1, _ST // 2, pbody, 0)
            for b in range(2):
                edgev = bufs[b][0]
                sl = bufs[b][4]
                erow = s * _ST + (_ST - 1)
                pltpu.make_async_copy(edge_hbm.at[erow], edgev, sl).wait()
                drain(b)
            plsc.subcore_barrier()

            woff = pl.multiple_of(s * _RB, 8)

            @pl.when(c == 0)
            def _():
                pltpu.sync_copy(acc.at[pl.ds(woff, _RB)],
                                outa.at[j, pl.ds(woff, _RB)])

            @pl.when(c == 1)
            def _():
                pltpu.sync_copy(acc.at[pl.ds(woff, _RB)],
                                outb.at[j, pl.ds(woff, _RB)])
            plsc.subcore_barrier()

    return k


def _aggregate(g64, edge4, zeros_h):
    outa, outb = _make_agg()(g64.reshape(_N * _CG, 16), edge4, zeros_h)
    agg_gm = jnp.concatenate([outa, outb], axis=0)[:, :_N]
    return agg_gm.transpose(1, 0, 2).reshape(_N, 64)


def kernel(x, edge_index, W1, b1, W2, b2, W3, b3, W4, b4):
    w4p = jnp.pad(W4, ((0, 0), (0, 11)))
    b4p = jnp.pad(b4, (0, 11))

    src = edge_index[0]
    dst = edge_index[1]
    padlen = _EPAD - _E
    ar = jnp.arange(padlen, dtype=jnp.int32)
    src1d = jnp.concatenate([src, ar % _N]).reshape(_EPAD // _BLK, 2, 128)
    dst1d = jnp.concatenate([dst, _N + (ar % _PADN)]).reshape(_EPAD // _BLK, 2, 128)
    edge4 = jnp.concatenate([src1d, dst1d], axis=1)

    zeros_h = jnp.zeros((_ZCH, 16), jnp.float32)
    ones_g = jnp.ones((_N, 64), jnp.float32)

    deg = _aggregate(ones_g, edge4, zeros_h)[:, 0:16]

    h1, g1, dis = _layer1(x, W1, deg)
    agg1 = _aggregate(g1, edge4, zeros_h)[:, :32]
    h2, g2 = _layer_mid(agg1, h1, dis, b1.reshape(1, -1), W2)
    agg2 = _aggregate(g2, edge4, zeros_h)
    h3, g3a, g3b = _layer_mid(agg2, h2, dis, b2.reshape(1, -1), W3)
    agg3 = jnp.concatenate(
        [_aggregate(g3a, edge4, zeros_h),
         _aggregate(g3b, edge4, zeros_h)], axis=1)
    h4, g4 = _layer_mid(agg3, h3, dis, b3.reshape(1, -1), w4p)
    agg4 = _aggregate(g4, edge4, zeros_h)[:, :32]
    out = _layer_fin(agg4, h4, dis, b4p.reshape(1, -1))
    return out[:, :21]
